# baseline (speedup 1.0000x reference)
import functools

import jax
import jax.numpy as jnp
from jax.experimental import pallas as pl
from jax.experimental.pallas import tpu as pltpu

_VMEM_LIMIT = 48 * 1024 * 1024
_NEG = -1e30


def _shift_cat(X, k):
    r, c = X.shape
    parts = [X]
    for s in range(1, k):
        parts.append(jnp.concatenate(
            [X[s:], jnp.zeros((s, c), X.dtype)], axis=0))
    return jnp.concatenate(parts, axis=1)


def _col_mask(rows, w, valid):
    col = jax.lax.broadcasted_iota(jnp.int32, (rows, 1), 0) % w
    return (col < valid).astype(jnp.float32)


def _body(x_ref, w1_ref, w2_ref, dsw_ref, b1_ref, g1_ref, be1_ref,
          b2_ref, g2_ref, be2_ref, ca1_ref, ca2_ref, sa_a_ref, sa_m_ref,
          o_ref, x5_s, tx3_s, acc1_s, acc2_s, acc3_s, ug_s, apad_s, mpad_s,
          *, n, h, w, eps):
    s = pl.program_id(0)
    h1, ho, wo = h - 2, h - 4, w - 4
    cin = x_ref.shape[1]
    cout = o_ref.shape[1]

    @pl.when(s == 0)
    def _init():
        xv = jnp.transpose(x_ref[...].astype(jnp.bfloat16), (0, 2, 1))
        for i in range(n):
            x5_s[i] = _shift_cat(xv[i], 5)
            acc1_s[i] = jnp.zeros((h1 * w, cout), jnp.float32)
            acc2_s[i] = jnp.zeros((ho * w, cout), jnp.float32)
            acc3_s[i] = jnp.zeros((ho * w, cout), jnp.float32)

    @pl.when(s < 3)
    def _conv1_tap():
        wslab = w1_ref[0].astype(jnp.bfloat16).reshape(3 * cin, cout)
        for i in range(n):
            lhs = x5_s[i, pl.ds(s * w, h1 * w), 0:3 * cin]
            acc1_s[i] = acc1_s[i] + jnp.dot(
                lhs, wslab, preferred_element_type=jnp.float32)

    @pl.when(s == 2)
    def _bn1():
        m1 = _col_mask(h1 * w, w, w - 2)
        cnt = float(n * h1 * (w - 2))
        accs = [acc1_s[i] + b1_ref[0] for i in range(n)]
        su = sum(jnp.sum(a * m1, axis=0) for a in accs)
        q = sum(jnp.sum(a * a * m1, axis=0) for a in accs)
        mean = su / cnt
        var = q / cnt - mean * mean
        scale = g1_ref[0] * jax.lax.rsqrt(var + eps)
        shift = be1_ref[0] - mean * scale
        for i in range(n):
            t1 = (jnp.maximum(accs[i] * scale + shift, 0.0) * m1) \
                .astype(jnp.bfloat16)
            tx3_s[i] = _shift_cat(t1, 3)

    @pl.when((s >= 3) & (s < 6))
    def _conv2_tap():
        wslab = w2_ref[0].astype(jnp.bfloat16).reshape(3 * cout, cout)
        for i in range(n):
            lhs = tx3_s[i, pl.ds((s - 3) * w, ho * w), :]
            acc2_s[i] = acc2_s[i] + jnp.dot(
                lhs, wslab, preferred_element_type=jnp.float32)

    @pl.when(s == 5)
    def _bn2_attention():
        m = _col_mask(ho * w, w, wo)
        cnt = float(n * ho * wo)
        accs = [acc2_s[i] + b2_ref[0] for i in range(n)]
        su = sum(jnp.sum(a * m, axis=0) for a in accs)
        q = sum(jnp.sum(a * a * m, axis=0) for a in accs)
        mean = su / cnt
        var = q / cnt - mean * mean
        scale = g2_ref[0] * jax.lax.rsqrt(var + eps)
        shift = be2_ref[0] - mean * scale

        ybs = [(accs[i] * scale + shift) * m for i in range(n)]
        avgs = [jnp.sum(yb, axis=0) / float(ho * wo) for yb in ybs]
        maxs = [jnp.max(jnp.where(m > 0, yb, _NEG), axis=0) for yb in ybs]
        v = jnp.stack(avgs + maxs, axis=0)
        hmid = jnp.maximum(jnp.dot(v, ca1_ref[...],
                                   preferred_element_type=jnp.float32), 0.0)
        o2 = jnp.dot(hmid, ca2_ref[...], preferred_element_type=jnp.float32)
        att = jax.nn.sigmoid(o2[:n] + o2[n:])

        for i in range(n):
            u3 = ybs[i].reshape(ho, w, cout) * att[i][None, None, :]
            amap = jnp.mean(u3, axis=-1)
            mmap = jnp.max(u3, axis=-1)
            apad_s[...] = jnp.zeros(apad_s.shape, jnp.float32)
            mpad_s[...] = jnp.zeros(mpad_s.shape, jnp.float32)
            apad_s[3:3 + ho, 3:3 + wo] = amap[:, :wo]
            mpad_s[3:3 + ho, 3:3 + wo] = mmap[:, :wo]
            logits = jnp.zeros((ho, wo), jnp.float32)
            for dh in range(7):
                for dw in range(7):
                    logits = logits + sa_a_ref[dh, dw] * \
                        apad_s[dh:dh + ho, dw:dw + wo]
                    logits = logits + sa_m_ref[dh, dw] * \
                        mpad_s[dh:dh + ho, dw:dw + wo]
            gate = jnp.concatenate(
                [jax.nn.sigmoid(logits),
                 jnp.zeros((ho, w - wo), jnp.float32)], axis=1)
            ug_s[i] = (u3 * gate[:, :, None]).reshape(ho * w, cout)

    @pl.when(s >= 6)
    def _ds_tap():
        wslab = dsw_ref[0].astype(jnp.bfloat16).reshape(5 * cin, cout)
        for i in range(n):
            lhs = x5_s[i, pl.ds((s - 6) * w, ho * w), :]
            acc3_s[i] = acc3_s[i] + jnp.dot(
                lhs, wslab, preferred_element_type=jnp.float32)

    @pl.when(s == 10)
    def _finish():
        m = _col_mask(ho * w, w, wo)
        for i in range(n):
            o = jnp.maximum(ug_s[i] + acc3_s[i] * m, 0.0)
            oc = o.reshape(ho, w, cout)[:, :wo, :].reshape(ho * wo, cout)
            o_ref[i] = jnp.transpose(oc, (1, 0))


def kernel(x, conv1_w, conv1_b, bn1_g, bn1_b, conv2_w, conv2_b, bn2_g,
           bn2_b, ca_w1, ca_w2, sa_w, ds_w):
    eps = 1e-5
    n, cin, h, w = x.shape
    cout = conv1_w.shape[3]
    h1 = h - 2
    ho, wo = h - 4, w - 4

    xh = x.reshape(n, cin, h * w)
    sa_a = sa_w[:, :, 0, 0]
    sa_m = sa_w[:, :, 1, 0]

    def full(shape):
        nd = len(shape)
        return pl.BlockSpec(shape, lambda s, _nd=nd: (0,) * _nd)

    flops = 2 * n * h1 * w * 9 * cin * cout \
        + 2 * n * ho * w * 9 * cout * cout \
        + 2 * n * ho * w * 25 * cin * cout
    bytes_accessed = 4 * (xh.size + conv1_w.size + conv2_w.size + ds_w.size
                          + n * ho * wo * cout)

    out = pl.pallas_call(
        functools.partial(_body, n=n, h=h, w=w, eps=eps),
        out_shape=jax.ShapeDtypeStruct((n, cout, ho * wo), jnp.float32),
        grid=(11,),
        in_specs=[
            full(xh.shape),
            pl.BlockSpec((1, 3, cin, cout),
                         lambda s: (jnp.minimum(s, 2), 0, 0, 0)),
            pl.BlockSpec((1, 3, cout, cout),
                         lambda s: (jnp.clip(s - 3, 0, 2), 0, 0, 0)),
            pl.BlockSpec((1, 5, cin, cout),
                         lambda s: (jnp.clip(s - 6, 0, 4), 0, 0, 0)),
            full((1, cout)), full((1, cout)), full((1, cout)),
            full((1, cout)), full((1, cout)), full((1, cout)),
            full(ca_w1.shape), full(ca_w2.shape),
            pl.BlockSpec(memory_space=pltpu.MemorySpace.SMEM),
            pl.BlockSpec(memory_space=pltpu.MemorySpace.SMEM),
        ],
        out_specs=full((n, cout, ho * wo)),
        scratch_shapes=[
            pltpu.VMEM((n, h * w, 5 * cin), jnp.bfloat16),
            pltpu.VMEM((n, h1 * w, 3 * cout), jnp.bfloat16),
            pltpu.VMEM((n, h1 * w, cout), jnp.float32),
            pltpu.VMEM((n, ho * w, cout), jnp.float32),
            pltpu.VMEM((n, ho * w, cout), jnp.float32),
            pltpu.VMEM((n, ho * w, cout), jnp.float32),
            pltpu.VMEM((ho + 6, wo + 6), jnp.float32),
            pltpu.VMEM((ho + 6, wo + 6), jnp.float32),
        ],
        compiler_params=pltpu.CompilerParams(
            dimension_semantics=("arbitrary",),
            vmem_limit_bytes=_VMEM_LIMIT),
        cost_estimate=pl.CostEstimate(
            flops=int(flops),
            transcendentals=int(n * (ho * wo + 2 * cout)),
            bytes_accessed=int(bytes_accessed)),
    )(xh, conv1_w, conv2_w, ds_w,
      conv1_b.reshape(1, cout), bn1_g.reshape(1, cout),
      bn1_b.reshape(1, cout), conv2_b.reshape(1, cout),
      bn2_g.reshape(1, cout), bn2_b.reshape(1, cout),
      ca_w1, ca_w2, sa_a, sa_m)

    return out.reshape(n, cout, ho, wo)


# speedup vs baseline: 1.1636x; 1.1636x over previous
import functools

import jax
import jax.numpy as jnp
from jax.experimental import pallas as pl
from jax.experimental.pallas import tpu as pltpu

_VMEM_LIMIT = 48 * 1024 * 1024
_NEG = -1e30


def _shift_cat(X, k):
    r, c = X.shape
    parts = [X]
    for s in range(1, k):
        parts.append(jnp.concatenate(
            [X[s:], jnp.zeros((s, c), X.dtype)], axis=0))
    return jnp.concatenate(parts, axis=1)


def _col_mask(rows, w, valid):
    col = jax.lax.broadcasted_iota(jnp.int32, (rows, 1), 0) % w
    return (col < valid).astype(jnp.float32)


def _k1_body(x_ref, w1_ref, b1_ref, g1_ref, be1_ref, dsw_ref,
             t1_ref, res_ref, *, h, w, eps):
    n, h1, _, cblk = t1_ref.shape
    ho = h - 4
    cin = x_ref.shape[1]

    xv = jnp.transpose(x_ref[...].astype(jnp.bfloat16), (0, 2, 1))
    x5 = [_shift_cat(xv[i], 5) for i in range(n)]

    w1 = w1_ref[...].astype(jnp.bfloat16)
    w1r = w1.reshape(9 * cin, cblk)
    accs = []
    for i in range(n):
        acc = jnp.zeros((h1 * w, cblk), jnp.float32)
        for dh in range(3):
            lhs = x5[i][dh * w: dh * w + h1 * w, : 3 * cin]
            acc = acc + jnp.dot(lhs, w1r[dh * 3 * cin:(dh + 1) * 3 * cin],
                                preferred_element_type=jnp.float32)
        accs.append(acc + b1_ref[0])
    m1 = _col_mask(h1 * w, w, w - 2)
    cnt = float(n * h1 * (w - 2))
    s = sum(jnp.sum(a * m1, axis=0) for a in accs)
    q = sum(jnp.sum(a * a * m1, axis=0) for a in accs)
    mean = s / cnt
    var = q / cnt - mean * mean
    scale = g1_ref[0] * jax.lax.rsqrt(var + eps)
    shift = be1_ref[0] - mean * scale
    for i in range(n):
        t1 = jnp.maximum(accs[i] * scale + shift, 0.0) * m1
        t1_ref[i] = t1.reshape(h1, w, cblk).astype(t1_ref.dtype)

    dsw = dsw_ref[...].astype(jnp.bfloat16)
    dsr = dsw.reshape(25 * cin, cblk)
    m2 = _col_mask(ho * w, w, w - 4)
    for i in range(n):
        acc = jnp.zeros((ho * w, cblk), jnp.float32)
        for dh in range(5):
            lhs = x5[i][dh * w: dh * w + ho * w, :]
            acc = acc + jnp.dot(lhs, dsr[dh * 5 * cin:(dh + 1) * 5 * cin],
                                preferred_element_type=jnp.float32)
        res_ref[i] = (acc * m2).astype(res_ref.dtype)


def _k2_body(t1_ref, w2_ref, b2_ref, g2_ref, be2_ref,
             y_ref, avg_ref, max_ref, *, eps):
    n, h1, w, c = t1_ref.shape
    ho = h1 - 2
    cblk = y_ref.shape[2]

    tv = t1_ref[...].reshape(n, h1 * w, c)
    w2 = w2_ref[...].astype(jnp.bfloat16)
    w2r = w2.reshape(9 * c, cblk)
    m = _col_mask(ho * w, w, w - 4)
    cnt = float(n * ho * (w - 4))

    accs = []
    for i in range(n):
        x3 = _shift_cat(tv[i], 3)
        acc = jnp.zeros((ho * w, cblk), jnp.float32)
        for dh in range(3):
            lhs = x3[dh * w: dh * w + ho * w]
            acc = acc + jnp.dot(lhs, w2r[dh * 3 * c:(dh + 1) * 3 * c],
                                preferred_element_type=jnp.float32)
        accs.append(acc + b2_ref[0])
    s = sum(jnp.sum(a * m, axis=0) for a in accs)
    q = sum(jnp.sum(a * a * m, axis=0) for a in accs)
    mean = s / cnt
    var = q / cnt - mean * mean
    scale = g2_ref[0] * jax.lax.rsqrt(var + eps)
    shift = be2_ref[0] - mean * scale
    for i in range(n):
        yb = (accs[i] * scale + shift) * m
        y_ref[i] = yb.astype(y_ref.dtype)
        avg_ref[i] = jnp.sum(yb, axis=0) / float(ho * (w - 4))
        max_ref[i] = jnp.max(jnp.where(m > 0, yb, _NEG), axis=0)


def _k3_body(y_ref, res_ref, avg_ref, max_ref, ca1_ref, ca2_ref,
             sa_a_ref, sa_m_ref, o_ref, apad_ref, mpad_ref, *, ho, w):
    _, sp, c = y_ref.shape
    wo = w - 4

    nb = avg_ref.shape[0]
    v = jnp.concatenate([avg_ref[...], max_ref[...]], axis=0)
    hmid = jnp.maximum(jnp.dot(v, ca1_ref[...],
                               preferred_element_type=jnp.float32), 0.0)
    o2 = jnp.dot(hmid, ca2_ref[...], preferred_element_type=jnp.float32)
    att_all = jax.nn.sigmoid(o2[:nb] + o2[nb:])
    sel = (jax.lax.broadcasted_iota(jnp.int32, (nb, 1), 0)
           == pl.program_id(0)).astype(jnp.float32)
    att = jnp.sum(att_all * sel, axis=0)

    u3 = y_ref[0].astype(jnp.float32).reshape(ho, w, c) \
        * att[None, None, :]

    amap = jnp.mean(u3, axis=-1)
    mmap = jnp.max(u3, axis=-1)
    apad_ref[...] = jnp.zeros(apad_ref.shape, jnp.float32)
    mpad_ref[...] = jnp.zeros(mpad_ref.shape, jnp.float32)
    apad_ref[3:3 + ho, 3:3 + wo] = amap[:, :wo]
    mpad_ref[3:3 + ho, 3:3 + wo] = mmap[:, :wo]

    logits = jnp.zeros((ho, wo), jnp.float32)
    for dh in range(7):
        for dw in range(7):
            logits = logits + sa_a_ref[dh, dw] * \
                apad_ref[dh:dh + ho, dw:dw + wo]
            logits = logits + sa_m_ref[dh, dw] * \
                mpad_ref[dh:dh + ho, dw:dw + wo]

    gate = jax.nn.sigmoid(logits)

    gate = jnp.concatenate(
        [gate, jnp.zeros((ho, w - wo), jnp.float32)], axis=1)
    res3 = res_ref[0].astype(jnp.float32).reshape(ho, w, c)
    o3 = jnp.maximum(u3 * gate[:, :, None] + res3, 0.0)
    oc = o3[:, :wo, :].reshape(ho * wo, c)
    o_ref[...] = jnp.transpose(oc, (1, 0)).reshape(1, c, ho * wo)


def kernel(x, conv1_w, conv1_b, bn1_g, bn1_b, conv2_w, conv2_b, bn2_g,
           bn2_b, ca_w1, ca_w2, sa_w, ds_w):
    eps = 1e-5
    n, cin, h, w = x.shape
    cout = conv1_w.shape[3]
    h1 = h - 2
    ho, wo = h - 4, w - 4
    cblk = min(128, cout)
    nblk = cout // cblk

    xh = x.reshape(n, cin, h * w)
    sa_a = sa_w[:, :, 0, 0]
    sa_m = sa_w[:, :, 1, 0]

    def rep(shape):
        nd = len(shape)
        return pl.BlockSpec(shape, lambda i, _nd=nd: (0,) * _nd)

    def wspec(shape):
        return pl.BlockSpec(shape[:3] + (cblk,), lambda i: (0, 0, 0, i))

    vspec = pl.BlockSpec((1, cblk), lambda i: (0, i))

    k1_flops = 2 * n * h1 * w * 9 * cin * cout \
        + 2 * n * ho * w * 25 * cin * cout
    t1, res = pl.pallas_call(
        functools.partial(_k1_body, h=h, w=w, eps=eps),
        out_shape=(
            jax.ShapeDtypeStruct((n, h1, w, cout), jnp.bfloat16),
            jax.ShapeDtypeStruct((n, ho * w, cout), jnp.bfloat16)),
        grid=(nblk,),
        in_specs=[rep(xh.shape), wspec(conv1_w.shape),
                  vspec, vspec, vspec, wspec(ds_w.shape)],
        out_specs=(pl.BlockSpec((n, h1, w, cblk), lambda i: (0, 0, 0, i)),
                   pl.BlockSpec((n, ho * w, cblk), lambda i: (0, 0, i))),
        compiler_params=pltpu.CompilerParams(
            dimension_semantics=("parallel",),
            vmem_limit_bytes=_VMEM_LIMIT),
        cost_estimate=pl.CostEstimate(
            flops=int(k1_flops), transcendentals=int(cout),
            bytes_accessed=int(4 * xh.size + 4 * conv1_w.size
                               + 4 * ds_w.size + 2 * n * h1 * w * cout
                               + 4 * n * ho * w * cout)),
    )(xh, conv1_w, conv1_b.reshape(1, cout), bn1_g.reshape(1, cout),
      bn1_b.reshape(1, cout), ds_w)

    k2_flops = 2 * n * ho * w * 9 * cout * cout
    y, avgp, maxp = pl.pallas_call(
        functools.partial(_k2_body, eps=eps),
        out_shape=(
            jax.ShapeDtypeStruct((n, ho * w, cout), jnp.bfloat16),
            jax.ShapeDtypeStruct((n, cout), jnp.float32),
            jax.ShapeDtypeStruct((n, cout), jnp.float32)),
        grid=(nblk,),
        in_specs=[rep(t1.shape), wspec(conv2_w.shape),
                  vspec, vspec, vspec],
        out_specs=(pl.BlockSpec((n, ho * w, cblk), lambda i: (0, 0, i)),
                   pl.BlockSpec((n, cblk), lambda i: (0, i)),
                   pl.BlockSpec((n, cblk), lambda i: (0, i))),
        compiler_params=pltpu.CompilerParams(
            dimension_semantics=("parallel",),
            vmem_limit_bytes=_VMEM_LIMIT),
        cost_estimate=pl.CostEstimate(
            flops=int(k2_flops), transcendentals=int(cout),
            bytes_accessed=int(2 * t1.size + 4 * conv2_w.size
                               + 4 * n * ho * w * cout)),
    )(t1, conv2_w, conv2_b.reshape(1, cout), bn2_g.reshape(1, cout),
      bn2_b.reshape(1, cout))

    def per_n(shape):
        nd = len(shape)
        return pl.BlockSpec((1,) + shape[1:],
                            lambda j, _nd=nd: (j,) + (0,) * (_nd - 1))

    out = pl.pallas_call(
        functools.partial(_k3_body, ho=ho, w=w),
        out_shape=jax.ShapeDtypeStruct((n, cout, ho * wo), jnp.float32),
        grid=(n,),
        in_specs=[per_n((n, ho * w, cout)), per_n((n, ho * w, cout)),
                  rep((n, cout)), rep((n, cout)),
                  rep(ca_w1.shape), rep(ca_w2.shape),
                  pl.BlockSpec(memory_space=pltpu.MemorySpace.SMEM),
                  pl.BlockSpec(memory_space=pltpu.MemorySpace.SMEM)],
        out_specs=per_n((n, cout, ho * wo)),
        scratch_shapes=[pltpu.VMEM((ho + 6, wo + 6), jnp.float32),
                        pltpu.VMEM((ho + 6, wo + 6), jnp.float32)],
        compiler_params=pltpu.CompilerParams(
            dimension_semantics=("parallel",),
            vmem_limit_bytes=_VMEM_LIMIT),
        cost_estimate=pl.CostEstimate(
            flops=int(20 * n * ho * w * cout),
            transcendentals=int(n * (ho * wo + 2 * cout)),
            bytes_accessed=int(4 * (3 * n * ho * w * cout))),
    )(y, res, avgp, maxp, ca_w1, ca_w2, sa_a, sa_m)

    return out.reshape(n, cout, ho, wo)
